# baseline (speedup 1.0000x reference)
import jax
import jax.numpy as jnp
from jax import lax
from jax.experimental import pallas as pl
from jax.experimental.pallas import tpu as pltpu

B, L, D = 4096, 200, 128
BBM = 128
NSTEPS = B // BBM
NSLOT = 4


def _body(ids_hbm, pos_hbm, memb_hbm, out_hbm, lab_hbm, mask_hbm,
          buf0, buf1, buf2, buf3, idsv0, idsv1, labv0, labv1,
          maskv0, maskv1, rowzv, rowzs, membv,
          insems, outsems, idssems, labsems, masksems, rowzsem, membsem):
    bufs = (buf0, buf1, buf2, buf3)
    idsv = (idsv0, idsv1)
    labv = (labv0, labv1)
    maskv = (maskv0, maskv1)

    pltpu.make_async_copy(memb_hbm, membv, membsem).start()
    pltpu.make_async_copy(memb_hbm, membv, membsem).wait()

    def in_cp(s, t):
        return pltpu.make_async_copy(
            pos_hbm.at[pl.ds(t * BBM, BBM)], bufs[s], insems.at[s])

    def out_cp(s, t):
        return pltpu.make_async_copy(
            bufs[s], out_hbm.at[pl.ds(t * BBM, BBM)], outsems.at[s])

    def ids_cp(p, t):
        return pltpu.make_async_copy(
            ids_hbm.at[pl.ds(t * BBM, BBM)], idsv[p], idssems.at[p])

    def lab_cp(p, t):
        return pltpu.make_async_copy(
            labv[p], lab_hbm.at[pl.ds(t * BBM, BBM)], labsems.at[p])

    def mask_cp(p, t):
        return pltpu.make_async_copy(
            maskv[p], mask_hbm.at[pl.ds(t * BBM, BBM)], masksems.at[p])

    for s in range(NSLOT):
        in_cp(s, s).start()
    ids_cp(0, 0).start()
    ids_cp(1, 1).start()

    lane = jax.lax.broadcasted_iota(jnp.int32, (BBM, L), 1)
    memb = membv[...]

    def step(q, s):
        t = NSLOT * q + s
        p = s % 2
        ids_cp(p, t).wait()
        ids = idsv[p][...]
        labels = jnp.where(lane == (L - 1), 0, jnp.roll(ids, -1, axis=1))

        @pl.when(t >= 2)
        def _wait_small():
            lab_cp(p, t - 2).wait()
            mask_cp(p, t - 2).wait()
        labv[p][...] = labels
        maskv[p][...] = jnp.where(labels != 0, 1, 0).astype(jnp.int8)
        lab_cp(p, t).start()
        mask_cp(p, t).start()

        @pl.when(t + 2 < NSTEPS)
        def _ids_next():
            ids_cp(p, t + 2).start()

        in_cp(s, t).wait()
        buf = bufs[s]
        buf[:, L - 1, :] = jnp.broadcast_to(memb, (BBM, D))

        zero = jnp.logical_and(labels == 0, lane < (L - 1))
        anyz = jnp.any(zero)

        @pl.when(anyz)
        def _slow():
            rowzv[...] = jnp.any(zero, axis=1, keepdims=True).astype(jnp.int32)
            pltpu.make_async_copy(rowzv, rowzs, rowzsem).start()
            pltpu.make_async_copy(rowzv, rowzs, rowzsem).wait()

            def rbody(b, _):
                @pl.when(rowzs[b, 0] != 0)
                def _fix():
                    labrow = labv[p][pl.ds(b, 1), :]
                    lab3 = jnp.transpose(labrow.reshape(1, 1, L), (0, 2, 1))
                    buf[pl.ds(b, 1)] = jnp.where(
                        lab3 != 0, buf[pl.ds(b, 1)], memb[None])
                return 0
            lax.fori_loop(0, BBM, rbody, 0)

        out_cp(s, t).start()

        @pl.when(t + NSLOT < NSTEPS)
        def _refill():
            out_cp(s, t).wait()
            in_cp(s, t + NSLOT).start()

    def loop_body(q, _):
        for s in range(NSLOT):
            step(q, s)
        return 0

    lax.fori_loop(0, NSTEPS // NSLOT, loop_body, 0)
    for s in range(NSLOT):
        out_cp(s, NSTEPS - NSLOT + s).wait()
    for p in range(2):
        lab_cp(p, NSTEPS - 2 + p).wait()
        mask_cp(p, NSTEPS - 2 + p).wait()


def kernel(pos_emb, itemid_seq, training, masked_item_embedding):
    del training
    memb2 = masked_item_embedding.reshape(1, D)
    out, labels, mask = pl.pallas_call(
        _body,
        in_specs=[
            pl.BlockSpec(memory_space=pl.ANY),
            pl.BlockSpec(memory_space=pl.ANY),
            pl.BlockSpec(memory_space=pl.ANY),
        ],
        out_specs=[
            pl.BlockSpec(memory_space=pl.ANY),
            pl.BlockSpec(memory_space=pl.ANY),
            pl.BlockSpec(memory_space=pl.ANY),
        ],
        out_shape=[
            jax.ShapeDtypeStruct((B, L, D), jnp.float32),
            jax.ShapeDtypeStruct((B, L), jnp.int32),
            jax.ShapeDtypeStruct((B, L), jnp.int8),
        ],
        scratch_shapes=[
            pltpu.VMEM((BBM, L, D), jnp.float32),
            pltpu.VMEM((BBM, L, D), jnp.float32),
            pltpu.VMEM((BBM, L, D), jnp.float32),
            pltpu.VMEM((BBM, L, D), jnp.float32),
            pltpu.VMEM((BBM, L), jnp.int32),
            pltpu.VMEM((BBM, L), jnp.int32),
            pltpu.VMEM((BBM, L), jnp.int32),
            pltpu.VMEM((BBM, L), jnp.int32),
            pltpu.VMEM((BBM, L), jnp.int8),
            pltpu.VMEM((BBM, L), jnp.int8),
            pltpu.VMEM((BBM, 1), jnp.int32),
            pltpu.SMEM((BBM, 1), jnp.int32),
            pltpu.VMEM((1, D), jnp.float32),
            pltpu.SemaphoreType.DMA((NSLOT,)),
            pltpu.SemaphoreType.DMA((NSLOT,)),
            pltpu.SemaphoreType.DMA((2,)),
            pltpu.SemaphoreType.DMA((2,)),
            pltpu.SemaphoreType.DMA((2,)),
            pltpu.SemaphoreType.DMA,
            pltpu.SemaphoreType.DMA,
        ],
    )(itemid_seq, pos_emb, memb2)
    return (out, labels, mask.astype(jnp.bool_))
